# baseline (speedup 1.0000x reference)
import numpy as np
import jax
import jax.numpy as jnp
from jax.experimental import pallas as pl

DIM = 64
BASE = 10000.0
TRIPLE = 24

_INVF = np.power(BASE, -(np.arange(128) % 32) / 32.0).astype(np.float32)
_CONSTS = np.zeros((8, 128), dtype=np.float32)
_CONSTS[0, :] = _INVF
_CONSTS[1, :] = np.cos(16.0 * _INVF.astype(np.float64)).astype(np.float32)
_CONSTS[2, :] = np.sin(16.0 * _INVF.astype(np.float64)).astype(np.float32)


def _rope_kernel(const_ref, cos_ref, sin_ref):
    rows = cos_ref.shape[0]
    base_r = pl.program_id(0) * rows
    invf = const_ref[0, :]
    rot_c = const_ref[1, :]
    rot_s = const_ref[2, :]

    r = jax.lax.broadcasted_iota(jnp.int32, (TRIPLE, 128), 0)
    l = jax.lax.broadcasted_iota(jnp.int32, (TRIPLE, 128), 1)
    g = 2 * (r + base_r) + l // 64
    t = g // 3
    ident = (g - 3 * t) == 2
    phase = t.astype(jnp.float32) * invf
    c_seed = jnp.cos(phase)
    s_seed = jnp.sin(phase)

    one = jnp.float32(1.0)
    zero = jnp.float32(0.0)

    def body(i, carry):
        c, s = carry
        cos_ref[pl.ds(i * TRIPLE, TRIPLE), :] = jnp.where(ident, one, c)
        sin_ref[pl.ds(i * TRIPLE, TRIPLE), :] = jnp.where(ident, zero, s)
        cn = c * rot_c - s * rot_s
        sn = s * rot_c + c * rot_s
        return cn, sn

    jax.lax.fori_loop(0, rows // TRIPLE, body, (c_seed, s_seed))


def kernel(x):
    seq_len = x.shape[1]
    total_rows = seq_len * 3 * 64 // 128
    grid = 4 if total_rows % (4 * TRIPLE) == 0 else 1
    block_rows = total_rows // grid
    consts = jnp.asarray(_CONSTS)
    cos_f, sin_f = pl.pallas_call(
        _rope_kernel,
        grid=(grid,),
        in_specs=[pl.BlockSpec((8, 128), lambda i: (0, 0))],
        out_specs=[
            pl.BlockSpec((block_rows, 128), lambda i: (i, 0)),
            pl.BlockSpec((block_rows, 128), lambda i: (i, 0)),
        ],
        out_shape=[
            jax.ShapeDtypeStruct((total_rows, 128), jnp.float32),
            jax.ShapeDtypeStruct((total_rows, 128), jnp.float32),
        ],
    )(consts)
    shape = (1, seq_len, 3, 1, 64)
    return cos_f.reshape(shape), sin_f.reshape(shape)
